# Initial kernel scaffold; baseline (speedup 1.0000x reference)
import jax
import jax.numpy as jnp
from jax.experimental import pallas as pl


def kernel(x, edge_index, edge_attr, batch, parity_atoms, W_node, b_node, W_edge, b_edge, W_conv0, b_conv0, W_conv1, b_conv1, W_conv2, b_conv2, W_ffn0, b_ffn0, W_ffn1, b_ffn1):
    raise NotImplementedError("write your pallas kernel here")


# speedup vs baseline: 6.5455x; 6.5455x over previous
import functools

import jax
import jax.numpy as jnp
from jax import lax
from jax.experimental import pallas as pl
from jax.experimental.pallas import tpu as pltpu
from jax.experimental.pallas import tpu_sc as plsc

_NC = 2
_NS = 16
_L = 16
_NW = _NC * _NS
_G = 64


def _rsqrt_newton(x):
    xi = lax.bitcast_convert_type(x, jnp.int32)
    yi = jnp.int32(0x5F3759DF) - (xi >> 1)
    y = lax.bitcast_convert_type(yi, jnp.float32)
    for _ in range(3):
        y = y * (1.5 - 0.5 * x * y * y)
    return y


def _make_deg_dinv_kernel(NPAD, E, C=128):
    assert E % C == 0 and NPAD % (_NS * _L) == 0
    nch = E // C
    psb, pse = divmod(nch, _NS)
    pwb, pwe = divmod(nch, _NW)
    nps = NPAD // _NS
    mesh = plsc.VectorSubcoreMesh(core_axis_name="c", subcore_axis_name="s")

    @functools.partial(
        pl.kernel, mesh=mesh,
        compiler_params=pltpu.CompilerParams(needs_layout_passes=False),
        out_type=(jax.ShapeDtypeStruct((NPAD,), jnp.float32),
                  jax.ShapeDtypeStruct((E,), jnp.float32)),
        scratch_types=[
            pltpu.VMEM_SHARED((NPAD,), jnp.float32),
            pltpu.VMEM((C,), jnp.int32),
            pltpu.VMEM((C,), jnp.float32),
            pltpu.VMEM((NPAD,), jnp.float32),
            pltpu.VMEM((C,), jnp.float32),
        ],
    )
    def k(col_hbm, row_hbm, zero_hbm, dinv_hbm, dinvrow_hbm,
          deg_sh, idx_v, ones_v, dinv_v, obuf_v):
        cid = lax.axis_index("c")
        sid = lax.axis_index("s")
        wid = cid * _NS + sid

        def fill_ones(i, _):
            ones_v[pl.ds(i * _L, _L)] = jnp.full((_L,), 1.0, jnp.float32)
            return 0
        lax.fori_loop(0, C // _L, fill_ones, 0)

        pltpu.sync_copy(zero_hbm.at[pl.ds(sid * nps, nps)],
                        deg_sh.at[pl.ds(sid * nps, nps)])
        plsc.subcore_barrier()

        n_s = psb + jnp.where(sid < pse, 1, 0)
        base_s = (sid * psb + jnp.minimum(sid, pse)) * C

        def chunk1(i, _):
            off = base_s + i * C
            pltpu.sync_copy(col_hbm.at[pl.ds(off, C)], idx_v)
            pltpu.sync_copy(ones_v, deg_sh.at[idx_v], add=True)
            return 0
        lax.fori_loop(0, n_s, chunk1, 0)
        plsc.subcore_barrier()

        pltpu.sync_copy(deg_sh.at[pl.ds(sid * nps, nps)],
                        dinv_v.at[pl.ds(sid * nps, nps)])

        def conv(i, _):
            o = sid * nps + i * _L
            d = dinv_v[pl.ds(o, _L)]
            r = _rsqrt_newton(jnp.maximum(d, 1.0))
            dinv_v[pl.ds(o, _L)] = jnp.where(d > 0, r, 0.0)
            return 0
        lax.fori_loop(0, nps // _L, conv, 0)
        pltpu.sync_copy(dinv_v.at[pl.ds(sid * nps, nps)],
                        deg_sh.at[pl.ds(sid * nps, nps)])
        plsc.subcore_barrier()
        pltpu.sync_copy(deg_sh, dinv_v)

        @pl.when(cid == 0)
        def _():
            pltpu.sync_copy(dinv_v.at[pl.ds(sid * nps, nps)],
                            dinv_hbm.at[pl.ds(sid * nps, nps)])

        n_w = pwb + jnp.where(wid < pwe, 1, 0)
        base_w = (wid * pwb + jnp.minimum(wid, pwe)) * C

        def chunk3(i, _):
            off = base_w + i * C
            pltpu.sync_copy(row_hbm.at[pl.ds(off, C)], idx_v)
            for j in range(C // _L):
                ids = idx_v[pl.ds(j * _L, _L)]
                obuf_v[pl.ds(j * _L, _L)] = plsc.load_gather(dinv_v, [ids])
            pltpu.sync_copy(obuf_v, dinvrow_hbm.at[pl.ds(off, C)])
            return 0
        lax.fori_loop(0, n_w, chunk3, 0)

    return k


def _make_edge_pass_kernel(NPAD, H, E, gather, C=128):
    assert E % C == 0 and NPAD % _NS == 0
    nch = E // C
    pwb, pwe = divmod(nch, _NW)
    nps = NPAD // _NS
    mesh = plsc.VectorSubcoreMesh(core_axis_name="c", subcore_axis_name="s")

    scratch = [
        pltpu.VMEM_SHARED((NPAD, H), jnp.float32),
        pltpu.VMEM((C, H), jnp.float32),
        pltpu.VMEM((C,), jnp.int32),
        pltpu.VMEM((C,), jnp.int32),
        pltpu.SemaphoreType.DMA,
    ]

    @functools.partial(
        pl.kernel, mesh=mesh,
        compiler_params=pltpu.CompilerParams(needs_layout_passes=False),
        out_type=jax.ShapeDtypeStruct((_NC * NPAD, H), jnp.float32),
        scratch_types=scratch,
    )
    def k(src_hbm, row_hbm, col_hbm, zero_hbm, out_hbm,
          acc, buf, ci, ri, sem):
        cid = lax.axis_index("c")
        sid = lax.axis_index("s")
        wid = cid * _NS + sid
        pltpu.sync_copy(zero_hbm.at[pl.ds(sid * nps, nps), :],
                        acc.at[pl.ds(sid * nps, nps), :])
        plsc.subcore_barrier()

        n_w = pwb + jnp.where(wid < pwe, 1, 0)
        base_w = (wid * pwb + jnp.minimum(wid, pwe)) * C

        def chunk(i, _):
            off = base_w + i * C
            pltpu.sync_copy(col_hbm.at[pl.ds(off, C)], ci)
            if gather:
                pltpu.sync_copy(row_hbm.at[pl.ds(off, C)], ri)
                pltpu.async_copy(src_hbm.at[ri], buf, sem).wait()
            else:
                pltpu.sync_copy(src_hbm.at[pl.ds(off, C), :], buf)
            pltpu.sync_copy(buf, acc.at[ci], add=True)
            return 0
        lax.fori_loop(0, n_w, chunk, 0)
        plsc.subcore_barrier()
        pltpu.sync_copy(acc.at[pl.ds(sid * nps, nps), :],
                        out_hbm.at[pl.ds(cid * NPAD + sid * nps, nps), :])

    return k


def _tc_init(xp, Wn, bn, W0, b0, dinv2):
    NPAD, DF = xp.shape
    H = Wn.shape[1]
    BR = 1024

    def body(x_ref, wn_ref, bn_ref, w0_ref, b0_ref, dv_ref, xh_ref, hs_ref):
        xh = jnp.maximum(
            jnp.dot(x_ref[...], wn_ref[...],
                    preferred_element_type=jnp.float32, precision=lax.Precision.DEFAULT) + bn_ref[...], 0.0)
        xh_ref[...] = xh
        hs_ref[...] = dv_ref[...] * (
            jnp.dot(xh, w0_ref[...], preferred_element_type=jnp.float32, precision=lax.Precision.DEFAULT)
            + b0_ref[...])

    return pl.pallas_call(
        body,
        grid=(NPAD // BR,),
        in_specs=[
            pl.BlockSpec((BR, DF), lambda i: (i, 0)),
            pl.BlockSpec((DF, H), lambda i: (0, 0)),
            pl.BlockSpec((1, H), lambda i: (0, 0)),
            pl.BlockSpec((H, H), lambda i: (0, 0)),
            pl.BlockSpec((1, H), lambda i: (0, 0)),
            pl.BlockSpec((BR, 1), lambda i: (i, 0)),
        ],
        out_specs=[pl.BlockSpec((BR, H), lambda i: (i, 0))] * 2,
        out_shape=[jax.ShapeDtypeStruct((NPAD, H), jnp.float32)] * 2,
    )(xp, Wn, bn, W0, b0, dinv2)


def _tc_edge(ea, We, be, dinv_row2):
    E, DE = ea.shape
    H = We.shape[1]
    BR = 2560

    def body(ea_ref, we_ref, be_ref, dr_ref, out_ref):
        v = jnp.maximum(
            jnp.dot(ea_ref[...], we_ref[...],
                    preferred_element_type=jnp.float32, precision=lax.Precision.DEFAULT) + be_ref[...], 0.0)
        out_ref[...] = dr_ref[...] * v

    return pl.pallas_call(
        body,
        grid=(E // BR,),
        in_specs=[
            pl.BlockSpec((BR, DE), lambda i: (i, 0)),
            pl.BlockSpec((DE, H), lambda i: (0, 0)),
            pl.BlockSpec((1, H), lambda i: (0, 0)),
            pl.BlockSpec((BR, 1), lambda i: (i, 0)),
        ],
        out_specs=pl.BlockSpec((BR, H), lambda i: (i, 0)),
        out_shape=jax.ShapeDtypeStruct((E, H), jnp.float32),
    )(ea, We, be, dinv_row2)


def _tc_layer(r0, r1, s0, s1, xh, dinv2, W, b):
    NPAD, H = xh.shape
    BR = 1024

    def body(r0_ref, r1_ref, s0_ref, s1_ref, xh_ref, dv_ref, w_ref, b_ref,
             xh1_ref, hs_ref):
        tot = r0_ref[...] + r1_ref[...] + s0_ref[...] + s1_ref[...]
        xh1 = dv_ref[...] * tot + xh_ref[...]
        xh1_ref[...] = xh1
        hs_ref[...] = dv_ref[...] * (
            jnp.dot(xh1, w_ref[...], preferred_element_type=jnp.float32, precision=lax.Precision.DEFAULT)
            + b_ref[...])

    blk = pl.BlockSpec((BR, H), lambda i: (i, 0))
    return pl.pallas_call(
        body,
        grid=(NPAD // BR,),
        in_specs=[blk, blk, blk, blk, blk,
                  pl.BlockSpec((BR, 1), lambda i: (i, 0)),
                  pl.BlockSpec((H, H), lambda i: (0, 0)),
                  pl.BlockSpec((1, H), lambda i: (0, 0))],
        out_specs=[blk, blk],
        out_shape=[jax.ShapeDtypeStruct((NPAD, H), jnp.float32)] * 2,
    )(r0, r1, s0, s1, xh, dinv2, W, b)


def _tc_final(r0, r1, s0, s1, xh, dinv2, batch2, Wf0, bf0, Wf1, bf1):
    NPAD, H = xh.shape
    BR = 1024
    grid = NPAD // BR

    def body(r0_ref, r1_ref, s0_ref, s1_ref, xh_ref, dv_ref, b_ref,
             wf0_ref, bf0_ref, wf1_ref, bf1_ref, out_ref, sums, counts):
        i = pl.program_id(0)

        @pl.when(i == 0)
        def _():
            sums[...] = jnp.zeros_like(sums)
            counts[...] = jnp.zeros_like(counts)

        tot = r0_ref[...] + r1_ref[...] + s0_ref[...] + s1_ref[...]
        xh3 = dv_ref[...] * tot + xh_ref[...]
        gidx = lax.broadcasted_iota(jnp.int32, (1, _G), 1)
        onehot = (b_ref[...] == gidx).astype(jnp.float32)
        sums[...] += lax.dot_general(
            onehot, xh3, (((0,), (0,)), ((), ())),
            preferred_element_type=jnp.float32, precision=lax.Precision.HIGHEST)
        counts[...] += lax.dot_general(
            onehot, jnp.ones((onehot.shape[0], 1), jnp.float32),
            (((0,), (0,)), ((), ())), preferred_element_type=jnp.float32, precision=lax.Precision.DEFAULT)

        @pl.when(i == grid - 1)
        def _():
            pooled = sums[...] / jnp.maximum(counts[...], 1.0)
            hidden = jnp.maximum(
                jnp.dot(pooled, wf0_ref[...],
                        preferred_element_type=jnp.float32, precision=lax.Precision.DEFAULT) + bf0_ref[...],
                0.0)
            out_ref[...] = jnp.dot(
                hidden, wf1_ref[...],
                preferred_element_type=jnp.float32, precision=lax.Precision.DEFAULT) + bf1_ref[...]

    blk = pl.BlockSpec((BR, H), lambda i: (i, 0))
    return pl.pallas_call(
        body,
        grid=(grid,),
        in_specs=[blk, blk, blk, blk, blk,
                  pl.BlockSpec((BR, 1), lambda i: (i, 0)),
                  pl.BlockSpec((BR, 1), lambda i: (i, 0)),
                  pl.BlockSpec((H, H), lambda i: (0, 0)),
                  pl.BlockSpec((1, H), lambda i: (0, 0)),
                  pl.BlockSpec((H, 1), lambda i: (0, 0)),
                  pl.BlockSpec((1, 1), lambda i: (0, 0))],
        out_specs=pl.BlockSpec((_G, 1), lambda i: (0, 0)),
        out_shape=jax.ShapeDtypeStruct((_G, 1), jnp.float32),
        scratch_shapes=[pltpu.VMEM((_G, H), jnp.float32),
                        pltpu.VMEM((_G, 1), jnp.float32)],
    )(r0, r1, s0, s1, xh, dinv2, batch2, Wf0, bf0, Wf1, bf1)


def kernel(x, edge_index, edge_attr, batch, parity_atoms,
           W_node, b_node, W_edge, b_edge,
           W_conv0, b_conv0, W_conv1, b_conv1, W_conv2, b_conv2,
           W_ffn0, b_ffn0, W_ffn1, b_ffn1):
    N, DF = x.shape
    E = edge_index.shape[1]
    H = W_node.shape[1]
    NPAD = ((N + _NS * _NW - 1) // (_NS * _NW)) * (_NS * _NW)

    row = edge_index[0]
    col = edge_index[1]
    xp = jnp.pad(x, ((0, NPAD - N), (0, 0)))
    batchp = jnp.pad(batch, (0, NPAD - N), constant_values=_G)[:, None]
    zeros_n = jnp.zeros((NPAD,), jnp.float32)
    zeros_nh = jnp.zeros((NPAD, H), jnp.float32)

    deg_k = _make_deg_dinv_kernel(NPAD, E)
    gath_k = _make_edge_pass_kernel(NPAD, H, E, gather=True)
    lin_k = _make_edge_pass_kernel(NPAD, H, E, gather=False)

    dinv, dinv_row = deg_k(col, row, zeros_n)
    dinv2 = dinv[:, None]

    xh, hs = _tc_init(xp, W_node, b_node.reshape(1, H),
                      W_conv0, b_conv0.reshape(1, H), dinv2)
    ea2 = _tc_edge(edge_attr, W_edge, b_edge.reshape(1, H), dinv_row[:, None])

    S = lin_k(ea2, row, col, zeros_nh)
    s0, s1 = S[:NPAD], S[NPAD:]

    for Wl, bl in ((W_conv1, b_conv1), (W_conv2, b_conv2)):
        r = gath_k(hs, row, col, zeros_nh)
        xh, hs = _tc_layer(r[:NPAD], r[NPAD:], s0, s1, xh, dinv2,
                           Wl, bl.reshape(1, H))

    r = gath_k(hs, row, col, zeros_nh)
    out = _tc_final(r[:NPAD], r[NPAD:], s0, s1, xh, dinv2, batchp,
                    W_ffn0, b_ffn0.reshape(1, H),
                    W_ffn1, b_ffn1.reshape(1, 1))
    return out
